# baseline (speedup 1.0000x reference)
import jax
import jax.numpy as jnp
from jax import lax
from jax.experimental import pallas as pl
from jax.experimental.pallas import tpu as pltpu
from jax.experimental.pallas import tpu_sc as plsc

K_TOP_N = 8
SEQ = 8192
CH = 64
NC, NS, LANES = 2, 16, 16
NW = NC * NS
CGS = CH // LANES
CHUNK = 256
NCHUNK = SEQ // CHUNK
GRP = 8
NGRP = CHUNK // GRP
OUT_W = CH * K_TOP_N
B_SC = 96

SORT8 = ((0, 1), (2, 3), (4, 5), (6, 7),
         (0, 2), (1, 3), (4, 6), (5, 7),
         (1, 2), (5, 6),
         (0, 4), (1, 5), (2, 6), (3, 7),
         (2, 4), (3, 5),
         (1, 2), (3, 4), (5, 6))
BITONIC8 = ((0, 4), (1, 5), (2, 6), (3, 7),
            (0, 2), (1, 3), (4, 6), (5, 7),
            (0, 1), (2, 3), (4, 5), (6, 7))


def _net(v, pairs):
    for (i, j) in pairs:
        hi = jnp.maximum(v[i], v[j])
        lo = jnp.minimum(v[i], v[j])
        v[i], v[j] = hi, lo
    return v


def _sc_body(x_hbm, out_hbm, buf0, buf1, stage, sem0, sem1):
    wid = lax.axis_index("s") * NC + lax.axis_index("c")
    bpw = B_SC // NW
    neg_inf = jnp.full((LANES,), -jnp.inf, jnp.float32)

    def process(buf, T32):
        @plsc.parallel_loop(0, NGRP, unroll=2, carry=tuple(T32))
        def grp_loop(g, Tflat):
            s = g * GRP
            out = []
            for cg in range(CGS):
                T = list(Tflat[cg * K_TOP_N:(cg + 1) * K_TOP_N])
                v = [buf[s + j, pl.ds(cg * LANES, LANES)] for j in range(GRP)]
                v = _net(v, SORT8)
                m = [jnp.maximum(T[i], v[K_TOP_N - 1 - i])
                     for i in range(K_TOP_N)]
                out += _net(m, BITONIC8)
            return tuple(out)

        return grp_loop

    def chunk_src(b, ci):
        return x_hbm.at[b, pl.ds(ci * CHUNK, CHUNK), :]

    def batch_body(i, _carry):
        b = wid * bpw + i

        pltpu.make_async_copy(chunk_src(b, 0), buf0, sem0).start()

        def pair_body(pi, T32):
            ci = pi * 2
            pltpu.make_async_copy(chunk_src(b, ci), buf0, sem0).wait()
            pltpu.make_async_copy(chunk_src(b, ci + 1), buf1, sem1).start()
            T32 = process(buf0, T32)

            @pl.when(pi < NCHUNK // 2 - 1)
            def _():
                pltpu.make_async_copy(chunk_src(b, ci + 2), buf0, sem0).start()

            pltpu.make_async_copy(chunk_src(b, ci + 1), buf1, sem1).wait()
            T32 = process(buf1, T32)
            return T32

        T32 = lax.fori_loop(0, NCHUNK // 2, pair_body,
                            (neg_inf,) * (CGS * K_TOP_N))
        lane = lax.iota(jnp.int32, LANES)
        for cg in range(CGS):
            for j in range(K_TOP_N):
                idx = lane * K_TOP_N + (cg * LANES * K_TOP_N + j)
                plsc.store_scatter(stage, [idx], T32[cg * K_TOP_N + j])
        pltpu.sync_copy(stage, out_hbm.at[pl.ds(b * OUT_W, OUT_W)])
        return 0

    lax.fori_loop(0, bpw, batch_body, 0)


def _tc_body(x_ref, out_ref):
    a = x_ref[0].reshape(K_TOP_N, SEQ // K_TOP_N, CH)
    v = _net([a[i] for i in range(K_TOP_N)], SORT8)
    n = SEQ // K_TOP_N
    while n > 1:
        h = n // 2
        m = [jnp.maximum(v[i][:h], v[K_TOP_N - 1 - i][h:])
             for i in range(K_TOP_N)]
        v = _net(m, BITONIC8)
        n = h
    for i in range(K_TOP_N):
        out_ref[0, :, i] = v[i][0]


def kernel(inputs):
    x = inputs
    b = x.shape[0]
    b_tc = b - B_SC

    out_tc3 = pl.pallas_call(
        _tc_body,
        grid=(b_tc,),
        in_specs=[pl.BlockSpec((1, SEQ, CH), lambda i: (i + B_SC, 0, 0))],
        out_specs=pl.BlockSpec((1, CH, K_TOP_N), lambda i: (i, 0, 0)),
        out_shape=jax.ShapeDtypeStruct((b_tc, CH, K_TOP_N), jnp.float32),
    )(x)
    out_tc = out_tc3.reshape(b_tc, OUT_W)

    mesh = plsc.VectorSubcoreMesh(core_axis_name="c", subcore_axis_name="s")
    sc_fn = pl.kernel(
        _sc_body,
        out_type=jax.ShapeDtypeStruct((B_SC * OUT_W,), jnp.float32),
        mesh=mesh,
        scratch_types=[
            pltpu.VMEM((CHUNK, CH), jnp.float32),
            pltpu.VMEM((CHUNK, CH), jnp.float32),
            pltpu.VMEM((OUT_W,), jnp.float32),
            pltpu.SemaphoreType.DMA,
            pltpu.SemaphoreType.DMA,
        ],
        compiler_params=pltpu.CompilerParams(needs_layout_passes=False),
    )
    out_sc = sc_fn(x).reshape(B_SC, OUT_W)

    return jnp.concatenate([out_sc, out_tc], axis=0)


# speedup vs baseline: 1.0345x; 1.0345x over previous
import jax
import jax.numpy as jnp
from jax import lax
from jax.experimental import pallas as pl
from jax.experimental.pallas import tpu as pltpu
from jax.experimental.pallas import tpu_sc as plsc

K_TOP_N = 8
SEQ = 8192
CH = 64
NC, NS, LANES = 2, 16, 16
NW = NC * NS
CGS = CH // LANES
CHUNK = 256
NCHUNK = SEQ // CHUNK
GRP = 8
NGRP = CHUNK // GRP
OUT_W = CH * K_TOP_N
B_SC = 96

SORT8 = ((0, 1), (2, 3), (4, 5), (6, 7),
         (0, 2), (1, 3), (4, 6), (5, 7),
         (1, 2), (5, 6),
         (0, 4), (1, 5), (2, 6), (3, 7),
         (2, 4), (3, 5),
         (1, 2), (3, 4), (5, 6))
BITONIC8 = ((0, 4), (1, 5), (2, 6), (3, 7),
            (0, 2), (1, 3), (4, 6), (5, 7),
            (0, 1), (2, 3), (4, 5), (6, 7))


def _net(v, pairs):
    for (i, j) in pairs:
        hi = jnp.maximum(v[i], v[j])
        lo = jnp.minimum(v[i], v[j])
        v[i], v[j] = hi, lo
    return v


def _sc_body(x_hbm, out_hbm, buf0, buf1, stage, sem0, sem1):
    wid = lax.axis_index("s") * NC + lax.axis_index("c")
    bpw = B_SC // NW
    neg_inf = jnp.full((LANES,), -jnp.inf, jnp.float32)

    def process(buf, T32):
        def grp_body(g, Tflat):
            s = g * GRP
            out = []
            for cg in range(CGS):
                T = list(Tflat[cg * K_TOP_N:(cg + 1) * K_TOP_N])
                v = [buf[s + j, pl.ds(cg * LANES, LANES)] for j in range(GRP)]
                v = _net(v, SORT8)
                m = [jnp.maximum(T[i], v[K_TOP_N - 1 - i])
                     for i in range(K_TOP_N)]
                out += _net(m, BITONIC8)
            return tuple(out)

        return lax.fori_loop(0, NGRP, grp_body, tuple(T32))

    def chunk_src(b, ci):
        return x_hbm.at[b, pl.ds(ci * CHUNK, CHUNK), :]

    def batch_body(i, _carry):
        b = wid * bpw + i

        pltpu.make_async_copy(chunk_src(b, 0), buf0, sem0).start()

        def pair_body(pi, T32):
            ci = pi * 2
            pltpu.make_async_copy(chunk_src(b, ci), buf0, sem0).wait()
            pltpu.make_async_copy(chunk_src(b, ci + 1), buf1, sem1).start()
            T32 = process(buf0, T32)

            @pl.when(pi < NCHUNK // 2 - 1)
            def _():
                pltpu.make_async_copy(chunk_src(b, ci + 2), buf0, sem0).start()

            pltpu.make_async_copy(chunk_src(b, ci + 1), buf1, sem1).wait()
            T32 = process(buf1, T32)
            return T32

        T32 = lax.fori_loop(0, NCHUNK // 2, pair_body,
                            (neg_inf,) * (CGS * K_TOP_N))
        lane = lax.iota(jnp.int32, LANES)
        for cg in range(CGS):
            for j in range(K_TOP_N):
                idx = lane * K_TOP_N + (cg * LANES * K_TOP_N + j)
                plsc.store_scatter(stage, [idx], T32[cg * K_TOP_N + j])
        pltpu.sync_copy(stage, out_hbm.at[pl.ds(b * OUT_W, OUT_W)])
        return 0

    lax.fori_loop(0, bpw, batch_body, 0)


def _tc_body(x_ref, out_ref):
    a = x_ref[0].reshape(K_TOP_N, SEQ // K_TOP_N, CH)
    v = _net([a[i] for i in range(K_TOP_N)], SORT8)
    n = SEQ // K_TOP_N
    while n > 1:
        h = n // 2
        m = [jnp.maximum(v[i][:h], v[K_TOP_N - 1 - i][h:])
             for i in range(K_TOP_N)]
        v = _net(m, BITONIC8)
        n = h
    for i in range(K_TOP_N):
        out_ref[0, :, i] = v[i][0]


def kernel(inputs):
    x = inputs
    b = x.shape[0]
    b_tc = b - B_SC

    out_tc3 = pl.pallas_call(
        _tc_body,
        grid=(b_tc,),
        in_specs=[pl.BlockSpec((1, SEQ, CH), lambda i: (i + B_SC, 0, 0))],
        out_specs=pl.BlockSpec((1, CH, K_TOP_N), lambda i: (i, 0, 0)),
        out_shape=jax.ShapeDtypeStruct((b_tc, CH, K_TOP_N), jnp.float32),
    )(x)
    out_tc = out_tc3.reshape(b_tc, OUT_W)

    mesh = plsc.VectorSubcoreMesh(core_axis_name="c", subcore_axis_name="s")
    sc_fn = pl.kernel(
        _sc_body,
        out_type=jax.ShapeDtypeStruct((B_SC * OUT_W,), jnp.float32),
        mesh=mesh,
        scratch_types=[
            pltpu.VMEM((CHUNK, CH), jnp.float32),
            pltpu.VMEM((CHUNK, CH), jnp.float32),
            pltpu.VMEM((OUT_W,), jnp.float32),
            pltpu.SemaphoreType.DMA,
            pltpu.SemaphoreType.DMA,
        ],
        compiler_params=pltpu.CompilerParams(needs_layout_passes=False),
    )
    out_sc = sc_fn(x).reshape(B_SC, OUT_W)

    return jnp.concatenate([out_sc, out_tc], axis=0)


# speedup vs baseline: 1.0411x; 1.0065x over previous
import jax
import jax.numpy as jnp
from jax import lax
from jax.experimental import pallas as pl
from jax.experimental.pallas import tpu as pltpu
from jax.experimental.pallas import tpu_sc as plsc

K_TOP_N = 8
SEQ = 8192
CH = 64
NC, NS, LANES = 2, 16, 16
NW = NC * NS
CGS = CH // LANES
CHUNK = 256
NCHUNK = SEQ // CHUNK
GRP = 8
NGRP = CHUNK // GRP
OUT_W = CH * K_TOP_N
B_SC = 96

SORT8 = ((0, 1), (2, 3), (4, 5), (6, 7),
         (0, 2), (1, 3), (4, 6), (5, 7),
         (1, 2), (5, 6),
         (0, 4), (1, 5), (2, 6), (3, 7),
         (2, 4), (3, 5),
         (1, 2), (3, 4), (5, 6))
BITONIC8 = ((0, 4), (1, 5), (2, 6), (3, 7),
            (0, 2), (1, 3), (4, 6), (5, 7),
            (0, 1), (2, 3), (4, 5), (6, 7))


def _net(v, pairs):
    for (i, j) in pairs:
        hi = jnp.maximum(v[i], v[j])
        lo = jnp.minimum(v[i], v[j])
        v[i], v[j] = hi, lo
    return v


def _sc_body(x_hbm, out_hbm, buf0, buf1, stage, sem0, sem1):
    wid = lax.axis_index("s") * NC + lax.axis_index("c")
    bpw = B_SC // NW
    neg_inf = jnp.full((LANES,), -jnp.inf, jnp.float32)

    def process(buf, T32):
        def grp_body(g, Tflat):
            s = g * GRP
            out = []
            for cg in range(CGS):
                T = list(Tflat[cg * K_TOP_N:(cg + 1) * K_TOP_N])
                v = [buf[s + j, pl.ds(cg * LANES, LANES)] for j in range(GRP)]
                v = _net(v, SORT8)
                m = [jnp.maximum(T[i], v[K_TOP_N - 1 - i])
                     for i in range(K_TOP_N)]
                out += _net(m, BITONIC8)
            return tuple(out)

        return lax.fori_loop(0, NGRP, grp_body, tuple(T32))

    def chunk_src(b, ci):
        return x_hbm.at[b, pl.ds(ci * CHUNK, CHUNK), :]

    def batch_body(i, _carry):
        b = wid * bpw + i

        pltpu.make_async_copy(chunk_src(b, 0), buf0, sem0).start()

        def pair_body(pi, T32):
            ci = pi * 2
            pltpu.make_async_copy(chunk_src(b, ci), buf0, sem0).wait()
            pltpu.make_async_copy(chunk_src(b, ci + 1), buf1, sem1).start()
            T32 = process(buf0, T32)

            @pl.when(pi < NCHUNK // 2 - 1)
            def _():
                pltpu.make_async_copy(chunk_src(b, ci + 2), buf0, sem0).start()

            pltpu.make_async_copy(chunk_src(b, ci + 1), buf1, sem1).wait()
            T32 = process(buf1, T32)
            return T32

        T32 = lax.fori_loop(0, NCHUNK // 2, pair_body,
                            (neg_inf,) * (CGS * K_TOP_N))
        lane = lax.iota(jnp.int32, LANES)
        for cg in range(CGS):
            for j in range(K_TOP_N):
                idx = lane * K_TOP_N + (cg * LANES * K_TOP_N + j)
                plsc.store_scatter(stage, [idx], T32[cg * K_TOP_N + j])
        pltpu.sync_copy(stage, out_hbm.at[pl.ds(b * OUT_W, OUT_W)])
        return 0

    lax.fori_loop(0, bpw, batch_body, 0)


def _tc_body(x_ref, out_ref):
    a = x_ref[0].reshape(K_TOP_N, SEQ // K_TOP_N, CH)
    v = _net([a[i] for i in range(K_TOP_N)], SORT8)
    n = SEQ // K_TOP_N
    while n > 1:
        h = n // 2
        m = [jnp.maximum(v[i][:h], v[K_TOP_N - 1 - i][h:])
             for i in range(K_TOP_N)]
        v = _net(m, BITONIC8)
        n = h
    for i in range(K_TOP_N):
        out_ref[0, :, i] = v[i][0]


def kernel(inputs):
    x = inputs
    b = x.shape[0]
    b_tc = b - B_SC

    out_tc3 = pl.pallas_call(
        _tc_body,
        grid=(b_tc,),
        in_specs=[pl.BlockSpec((1, SEQ, CH), lambda i: (i + B_SC, 0, 0))],
        out_specs=pl.BlockSpec((1, CH, K_TOP_N), lambda i: (i, 0, 0)),
        out_shape=jax.ShapeDtypeStruct((b_tc, CH, K_TOP_N), jnp.float32),
    )(x)
    out_tc = out_tc3.reshape(b_tc, OUT_W)

    mesh = plsc.VectorSubcoreMesh(core_axis_name="c", subcore_axis_name="s")
    sc_fn = pl.kernel(
        _sc_body,
        out_type=jax.ShapeDtypeStruct((B_SC * OUT_W,), jnp.float32),
        mesh=mesh,
        scratch_types=[
            pltpu.VMEM((CHUNK, CH), jnp.float32),
            pltpu.VMEM((CHUNK, CH), jnp.float32),
            pltpu.VMEM((OUT_W,), jnp.float32),
            pltpu.SemaphoreType.DMA,
            pltpu.SemaphoreType.DMA,
        ],
        compiler_params=pltpu.CompilerParams(
            needs_layout_passes=False, has_side_effects=True),
    )
    out_sc = sc_fn(x).reshape(B_SC, OUT_W)

    return jnp.concatenate([out_sc, out_tc], axis=0)
